# baseline (speedup 1.0000x reference)
import functools

import jax
import jax.numpy as jnp
from jax import lax
from jax.experimental import pallas as pl
from jax.experimental.pallas import tpu as pltpu
from jax.experimental.pallas import tpu_sc as plsc

B = 16384
C = 105
P = 9
NW = 32
BW = B // NW
NV = BW // 16
NROWS = C // 8
BIG = 10 ** 9

_TAPS = [(0, 1, 3, 4), (1, 2, 4, 5), (3, 4, 6, 7), (4, 5, 7, 8)]


def _sc_body(x_hbm, cw_hbm, cb_hbm, out_hbm,
             inbuf, buf104, chbuf, cw_v, cb_v, mbuf, in_sem, out_sem):
    wid = lax.axis_index("s") * 2 + lax.axis_index("c")
    wb0 = wid * BW
    lane = lax.iota(jnp.int32, 16)

    def bcast(s):
        return lax.broadcast_in_dim(jnp.asarray(s, jnp.int32), (16,), ())

    pltpu.sync_copy(x_hbm.at[pl.ds(0, P), pl.ds(0, 1), pl.ds(0, 128)], mbuf)
    pad = lane >= P
    zero = bcast(0)
    x00 = plsc.load_gather(mbuf, [jnp.where(pad, zero, lane), zero, zero])
    mask = x00 == 0.0
    rowv = lane // 3
    colv = lane % 3
    src = []
    for k in range(P):
        dr = rowv - (k // 3)
        dc = colv - (k % 3)
        drow = dr * dr + dc * dc
        d = jnp.where(pad | mask, bcast(BIG), drow)
        mn = jnp.min(d)
        cand = jnp.where(d == bcast(mn), lane, bcast(16))
        src.append(jnp.min(cand))

    pltpu.sync_copy(cw_hbm, cw_v)
    pltpu.sync_copy(cb_hbm, cb_v)

    def issue_row(r, slot):
        for j in range(P):
            pltpu.async_copy(
                x_hbm.at[pl.ds(src[j], 1), pl.ds(8 * r, 8), pl.ds(wb0, BW)],
                inbuf.at[slot, j], in_sem)

    def wait_row(slot):
        for j in range(P):
            pltpu.make_async_copy(
                x_hbm.at[pl.ds(0, 1), pl.ds(0, 8), pl.ds(wb0, BW)],
                inbuf.at[slot, j], in_sem).wait()

    def wait_out():
        pltpu.make_async_copy(
            chbuf.at[pl.ds(0, 1)],
            out_hbm.at[pl.ds(0, 1), pl.ds(0, 1), pl.ds(wb0, BW)],
            out_sem).wait()

    def compute_channel(c, bufs, chsel):
        w = [plsc.load_gather(cw_v, [bcast(c * 4 + t)]) for t in range(4)]
        bias = plsc.load_gather(cb_v, [bcast(c)])

        @pl.when(c >= 2)
        def _():
            wait_out()

        @plsc.parallel_loop(0, NV, unroll=8)
        def _(v):
            p = [bufs(j, v) for j in range(P)]
            ys = []
            for taps in _TAPS:
                y = p[taps[0]] * w[0]
                for wi, pj in enumerate(taps[1:], start=1):
                    y = y + p[pj] * w[wi]
                ys.append(y)
            m = jnp.maximum(jnp.maximum(ys[0], ys[1]),
                            jnp.maximum(ys[2], ys[3])) + bias
            chbuf[chsel, 0, pl.ds(v * 16, 16)] = (
                jnp.maximum(m, 0.2 * m) + p[4])
        pltpu.async_copy(
            chbuf.at[pl.ds(chsel, 1)],
            out_hbm.at[pl.ds(c % 7, 1), pl.ds(c // 7, 1), pl.ds(wb0, BW)],
            out_sem)

    issue_row(0, 0)

    def row_body(r, carry):
        slot = r % 2
        wait_row(slot)

        @pl.when(r < NROWS - 1)
        def _():
            issue_row(r + 1, 1 - slot)

        for c8 in range(8):
            compute_channel(
                r * 8 + c8,
                lambda j, v, _s=slot, _c=c8:
                    inbuf[_s, j, 0, _c, pl.ds(v * 16, 16)],
                c8 % 2)
        return carry

    lax.fori_loop(0, NROWS, row_body, 0)

    for j in range(P):
        pltpu.sync_copy(
            x_hbm.at[pl.ds(src[j], 1), pl.ds(104, 1), pl.ds(wb0, BW)],
            buf104.at[j])
    compute_channel(104,
                    lambda j, v: buf104[j, 0, 0, pl.ds(v * 16, 16)], 0)

    wait_out()
    wait_out()


@jax.jit
def _sc_call(x3, cwflat, cb):
    mesh = plsc.VectorSubcoreMesh(core_axis_name="c", subcore_axis_name="s")
    f = functools.partial(
        pl.kernel, mesh=mesh,
        compiler_params=pltpu.CompilerParams(needs_layout_passes=False),
        out_type=jax.ShapeDtypeStruct((7, 15, B), jnp.float32),
        scratch_types=[
            pltpu.VMEM((2, P, 1, 8, BW), jnp.float32),
            pltpu.VMEM((P, 1, 1, BW), jnp.float32),
            pltpu.VMEM((2, 1, BW), jnp.float32),
            pltpu.VMEM((C * 4,), jnp.float32),
            pltpu.VMEM((C,), jnp.float32),
            pltpu.VMEM((P, 1, 128), jnp.float32),
            pltpu.SemaphoreType.DMA,
            pltpu.SemaphoreType.DMA,
        ],
    )(_sc_body)
    return f(x3, cwflat, cb)


def kernel(augmented_tensor_temp, conv_weight, conv_bias):
    x3 = augmented_tensor_temp.transpose(2, 3, 1, 0).reshape(P, C, B)
    cwflat = conv_weight.reshape(-1)
    out3 = _sc_call(x3, cwflat, conv_bias)
    return out3.transpose(2, 1, 0)


# speedup vs baseline: 1.0065x; 1.0065x over previous
import functools

import jax
import jax.numpy as jnp
from jax import lax
from jax.experimental import pallas as pl
from jax.experimental.pallas import tpu as pltpu
from jax.experimental.pallas import tpu_sc as plsc

B = 16384
C = 105
P = 9
NW = 32
BW = B // NW
NV = BW // 16
NROWS = C // 8
BIG = 10 ** 9

_TAPS = [(0, 1, 3, 4), (1, 2, 4, 5), (3, 4, 6, 7), (4, 5, 7, 8)]


def _sc_body(x_hbm, cw_hbm, cb_hbm, out_hbm,
             inbuf, buf104, chbuf, cw_v, cb_v, mbuf, in_sem, out_sem):
    wid = lax.axis_index("s") * 2 + lax.axis_index("c")
    wb0 = wid * BW
    lane = lax.iota(jnp.int32, 16)

    def bcast(s):
        return lax.broadcast_in_dim(jnp.asarray(s, jnp.int32), (16,), ())

    pltpu.sync_copy(x_hbm.at[pl.ds(0, P), pl.ds(0, 1), pl.ds(0, 128)], mbuf)
    pad = lane >= P
    zero = bcast(0)
    x00 = plsc.load_gather(mbuf, [jnp.where(pad, zero, lane), zero, zero])
    mask = x00 == 0.0
    rowv = lane // 3
    colv = lane % 3
    src = []
    for k in range(P):
        dr = rowv - (k // 3)
        dc = colv - (k % 3)
        drow = dr * dr + dc * dc
        d = jnp.where(pad | mask, bcast(BIG), drow)
        mn = jnp.min(d)
        cand = jnp.where(d == bcast(mn), lane, bcast(16))
        src.append(jnp.min(cand))

    pltpu.sync_copy(cw_hbm, cw_v)
    pltpu.sync_copy(cb_hbm, cb_v)

    def issue_row(r, slot):
        for j in range(P):
            pltpu.async_copy(
                x_hbm.at[pl.ds(src[j], 1), pl.ds(8 * r, 8), pl.ds(wb0, BW)],
                inbuf.at[slot, j], in_sem)

    def wait_row(slot):
        for j in range(P):
            pltpu.make_async_copy(
                x_hbm.at[pl.ds(0, 1), pl.ds(0, 8), pl.ds(wb0, BW)],
                inbuf.at[slot, j], in_sem).wait()

    def wait_out():
        pltpu.make_async_copy(
            chbuf.at[pl.ds(0, 1)],
            out_hbm.at[pl.ds(0, 1), pl.ds(0, 1), pl.ds(wb0, BW)],
            out_sem).wait()

    def compute_channel(c, bufs, chsel):
        w = [plsc.load_gather(cw_v, [bcast(c * 4 + t)]) for t in range(4)]
        bias = plsc.load_gather(cb_v, [bcast(c)])

        @pl.when(c >= 2)
        def _():
            wait_out()

        @plsc.parallel_loop(0, NV, unroll=4)
        def _(v):
            p = [bufs(j, v) for j in range(P)]
            ys = []
            for taps in _TAPS:
                y = p[taps[0]] * w[0]
                for wi, pj in enumerate(taps[1:], start=1):
                    y = y + p[pj] * w[wi]
                ys.append(y)
            m = jnp.maximum(jnp.maximum(ys[0], ys[1]),
                            jnp.maximum(ys[2], ys[3])) + bias
            chbuf[chsel, 0, pl.ds(v * 16, 16)] = (
                jnp.maximum(m, 0.2 * m) + p[4])
        pltpu.async_copy(
            chbuf.at[pl.ds(chsel, 1)],
            out_hbm.at[pl.ds(c % 7, 1), pl.ds(c // 7, 1), pl.ds(wb0, BW)],
            out_sem)

    def compute_row(r, slot):
        for c8 in range(8):
            compute_channel(
                r * 8 + c8,
                lambda j, v, _s=slot, _c=c8:
                    inbuf[_s, j, 0, _c, pl.ds(v * 16, 16)],
                c8 % 2)

    issue_row(0, 0)

    def row_body(t, carry):
        r = t * 2
        wait_row(0)
        issue_row(r + 1, 1)
        compute_row(r, 0)
        wait_row(1)
        issue_row(r + 2, 0)
        compute_row(r + 1, 1)
        return carry

    lax.fori_loop(0, (NROWS - 1) // 2, row_body, 0)
    wait_row(0)
    compute_row(NROWS - 1, 0)

    for j in range(P):
        pltpu.sync_copy(
            x_hbm.at[pl.ds(src[j], 1), pl.ds(104, 1), pl.ds(wb0, BW)],
            buf104.at[j])
    compute_channel(104,
                    lambda j, v: buf104[j, 0, 0, pl.ds(v * 16, 16)], 0)

    wait_out()
    wait_out()


@jax.jit
def _sc_call(x3, cwflat, cb):
    mesh = plsc.VectorSubcoreMesh(core_axis_name="c", subcore_axis_name="s")
    f = functools.partial(
        pl.kernel, mesh=mesh,
        compiler_params=pltpu.CompilerParams(needs_layout_passes=False),
        out_type=jax.ShapeDtypeStruct((7, 15, B), jnp.float32),
        scratch_types=[
            pltpu.VMEM((2, P, 1, 8, BW), jnp.float32),
            pltpu.VMEM((P, 1, 1, BW), jnp.float32),
            pltpu.VMEM((2, 1, BW), jnp.float32),
            pltpu.VMEM((C * 4,), jnp.float32),
            pltpu.VMEM((C,), jnp.float32),
            pltpu.VMEM((P, 1, 128), jnp.float32),
            pltpu.SemaphoreType.DMA,
            pltpu.SemaphoreType.DMA,
        ],
    )(_sc_body)
    return f(x3, cwflat, cb)


def kernel(augmented_tensor_temp, conv_weight, conv_bias):
    x3 = augmented_tensor_temp.transpose(2, 3, 1, 0).reshape(P, C, B)
    cwflat = conv_weight.reshape(-1)
    out3 = _sc_call(x3, cwflat, conv_bias)
    return out3.transpose(2, 1, 0)


# speedup vs baseline: 1.2139x; 1.2060x over previous
import functools

import jax
import jax.numpy as jnp
from jax import lax
from jax.experimental import pallas as pl
from jax.experimental.pallas import tpu as pltpu
from jax.experimental.pallas import tpu_sc as plsc

B = 16384
C = 105
P = 9
NW = 32
BW = B // NW
NV = BW // 16
NROWS = C // 8
BIG = 10 ** 9

_TAPS = [(0, 1, 3, 4), (1, 2, 4, 5), (3, 4, 6, 7), (4, 5, 7, 8)]


def _sc_body(x_hbm, cw_hbm, cb_hbm, out_hbm,
             inbuf, buf104, chbuf, cw_v, cb_v, mbuf, in_sem, out_sem):
    wid = lax.axis_index("s") * 2 + lax.axis_index("c")
    wb0 = wid * BW
    lane = lax.iota(jnp.int32, 16)

    def bcast(s):
        return lax.broadcast_in_dim(jnp.asarray(s, jnp.int32), (16,), ())

    pltpu.sync_copy(x_hbm.at[pl.ds(0, P), pl.ds(0, 1), pl.ds(0, 128)], mbuf)
    pad = lane >= P
    zero = bcast(0)
    x00 = plsc.load_gather(mbuf, [jnp.where(pad, zero, lane), zero, zero])
    mask = x00 == 0.0
    rowv = lane // 3
    colv = lane % 3
    src = []
    for k in range(P):
        dr = rowv - (k // 3)
        dc = colv - (k % 3)
        drow = dr * dr + dc * dc
        d = jnp.where(pad | mask, bcast(BIG), drow)
        mn = jnp.min(d)
        cand = jnp.where(d == bcast(mn), lane, bcast(16))
        src.append(jnp.min(cand))

    pltpu.sync_copy(cw_hbm, cw_v)
    pltpu.sync_copy(cb_hbm, cb_v)

    def issue_row(r, slot):
        for j in range(P):
            pltpu.async_copy(
                x_hbm.at[pl.ds(src[j], 1), pl.ds(8 * r, 8), pl.ds(wb0, BW)],
                inbuf.at[slot, j], in_sem)

    def wait_row(slot):
        for j in range(P):
            pltpu.make_async_copy(
                x_hbm.at[pl.ds(0, 1), pl.ds(0, 8), pl.ds(wb0, BW)],
                inbuf.at[slot, j], in_sem).wait()

    def wait_out():
        pltpu.make_async_copy(
            chbuf.at[pl.ds(0, 1)],
            out_hbm.at[pl.ds(0, 1), pl.ds(0, 1), pl.ds(wb0, BW)],
            out_sem).wait()

    def compute_channel(c, bufs, chsel):
        w = [plsc.load_gather(cw_v, [bcast(c * 4 + t)]) for t in range(4)]
        bias = plsc.load_gather(cb_v, [bcast(c)])

        @pl.when(c >= 2)
        def _():
            wait_out()

        @plsc.parallel_loop(0, NV, unroll=4)
        def _(v):
            p = [bufs(j, v) for j in range(P)]
            ys = []
            for taps in _TAPS:
                y = p[taps[0]] * w[0]
                for wi, pj in enumerate(taps[1:], start=1):
                    y = y + p[pj] * w[wi]
                ys.append(y)
            m = jnp.maximum(jnp.maximum(ys[0], ys[1]),
                            jnp.maximum(ys[2], ys[3])) + bias
            chbuf[chsel, 0, pl.ds(v * 16, 16)] = (
                jnp.maximum(m, 0.2 * m) + p[4])
        pltpu.async_copy(
            chbuf.at[pl.ds(chsel, 1)],
            out_hbm.at[pl.ds(c % 7, 1), pl.ds(c // 7, 1), pl.ds(wb0, BW)],
            out_sem)

    issue_row(0, 0)

    def row_body(r, carry):
        slot = r % 2
        wait_row(slot)

        @pl.when(r < NROWS - 1)
        def _():
            issue_row(r + 1, 1 - slot)

        for c8 in range(8):
            compute_channel(
                r * 8 + c8,
                lambda j, v, _s=slot, _c=c8:
                    inbuf[_s, j, 0, _c, pl.ds(v * 16, 16)],
                c8 % 2)
        return carry

    lax.fori_loop(0, NROWS, row_body, 0)

    for j in range(P):
        pltpu.sync_copy(
            x_hbm.at[pl.ds(src[j], 1), pl.ds(104, 1), pl.ds(wb0, BW)],
            buf104.at[j])
    compute_channel(104,
                    lambda j, v: buf104[j, 0, 0, pl.ds(v * 16, 16)], 0)

    wait_out()
    wait_out()


@jax.jit
def _sc_call(x3, cwflat, cb):
    mesh = plsc.VectorSubcoreMesh(core_axis_name="c", subcore_axis_name="s")
    f = functools.partial(
        pl.kernel, mesh=mesh,
        compiler_params=pltpu.CompilerParams(needs_layout_passes=False),
        out_type=jax.ShapeDtypeStruct((7, 15, B), jnp.float32),
        scratch_types=[
            pltpu.VMEM((2, P, 1, 8, BW), jnp.float32),
            pltpu.VMEM((P, 1, 1, BW), jnp.float32),
            pltpu.VMEM((2, 1, BW), jnp.float32),
            pltpu.VMEM((C * 4,), jnp.float32),
            pltpu.VMEM((C,), jnp.float32),
            pltpu.VMEM((P, 1, 128), jnp.float32),
            pltpu.SemaphoreType.DMA,
            pltpu.SemaphoreType.DMA,
        ],
    )(_sc_body)
    return f(x3, cwflat, cb)


def kernel(augmented_tensor_temp, conv_weight, conv_bias):
    x3 = augmented_tensor_temp.transpose(2, 3, 1, 0).reshape(P, C, B)
    cwflat = conv_weight.reshape(-1)
    out3 = _sc_call(x3, cwflat, conv_bias)
    return out3.transpose(2, 1, 0)


# speedup vs baseline: 1.2493x; 1.0291x over previous
import functools

import jax
import jax.numpy as jnp
from jax import lax
from jax.experimental import pallas as pl
from jax.experimental.pallas import tpu as pltpu
from jax.experimental.pallas import tpu_sc as plsc

B = 16384
C = 105
P = 9
NW = 32
BW = B // NW
NV = BW // 16
NROWS = C // 8
BIG = 10 ** 9

_TAPS = [(0, 1, 3, 4), (1, 2, 4, 5), (3, 4, 6, 7), (4, 5, 7, 8)]


def _sc_body(x_hbm, cw_hbm, cb_hbm, out_hbm,
             inbuf, buf104, chbuf, cw_v, cb_v, mbuf, in_sem, out_sem):
    wid = lax.axis_index("s") * 2 + lax.axis_index("c")
    wb0 = wid * BW
    lane = lax.iota(jnp.int32, 16)

    def bcast(s):
        return lax.broadcast_in_dim(jnp.asarray(s, jnp.int32), (16,), ())

    pltpu.sync_copy(x_hbm.at[pl.ds(0, P), pl.ds(0, 1), pl.ds(0, 128)], mbuf)
    pad = lane >= P
    zero = bcast(0)
    x00 = plsc.load_gather(mbuf, [jnp.where(pad, zero, lane), zero, zero])
    mask = x00 == 0.0
    rowv = lane // 3
    colv = lane % 3
    src = []
    for k in range(P):
        dr = rowv - (k // 3)
        dc = colv - (k % 3)
        drow = dr * dr + dc * dc
        d = jnp.where(pad | mask, bcast(BIG), drow)
        mn = jnp.min(d)
        cand = jnp.where(d == bcast(mn), lane, bcast(16))
        src.append(jnp.min(cand))

    pltpu.sync_copy(cw_hbm, cw_v)
    pltpu.sync_copy(cb_hbm, cb_v)

    def issue_row(r, slot):
        for j in range(P):
            pltpu.async_copy(
                x_hbm.at[pl.ds(src[j], 1), pl.ds(8 * r, 8), pl.ds(wb0, BW)],
                inbuf.at[slot, j], in_sem)

    def wait_row(slot):
        for j in range(P):
            pltpu.make_async_copy(
                x_hbm.at[pl.ds(0, 1), pl.ds(0, 8), pl.ds(wb0, BW)],
                inbuf.at[slot, j], in_sem).wait()

    def wait_out():
        pltpu.make_async_copy(
            chbuf.at[pl.ds(0, 1)],
            out_hbm.at[pl.ds(0, 1), pl.ds(0, 1), pl.ds(wb0, BW)],
            out_sem).wait()

    def compute_channel(c, bufs, chsel):
        w = [plsc.load_gather(cw_v, [bcast(c * 4 + t)]) for t in range(4)]
        bias = plsc.load_gather(cb_v, [bcast(c)])

        @pl.when(c >= 2)
        def _():
            wait_out()

        @plsc.parallel_loop(0, NV, unroll=2)
        def _(v):
            p = [bufs(j, v) for j in range(P)]
            ys = []
            for taps in _TAPS:
                y = p[taps[0]] * w[0]
                for wi, pj in enumerate(taps[1:], start=1):
                    y = y + p[pj] * w[wi]
                ys.append(y)
            m = jnp.maximum(jnp.maximum(ys[0], ys[1]),
                            jnp.maximum(ys[2], ys[3])) + bias
            chbuf[chsel, 0, pl.ds(v * 16, 16)] = (
                jnp.maximum(m, 0.2 * m) + p[4])
        pltpu.async_copy(
            chbuf.at[pl.ds(chsel, 1)],
            out_hbm.at[pl.ds(c % 7, 1), pl.ds(c // 7, 1), pl.ds(wb0, BW)],
            out_sem)

    issue_row(0, 0)

    def row_body(r, carry):
        slot = r % 2
        wait_row(slot)

        @pl.when(r < NROWS - 1)
        def _():
            issue_row(r + 1, 1 - slot)

        for c8 in range(8):
            compute_channel(
                r * 8 + c8,
                lambda j, v, _s=slot, _c=c8:
                    inbuf[_s, j, 0, _c, pl.ds(v * 16, 16)],
                c8 % 2)
        return carry

    lax.fori_loop(0, NROWS, row_body, 0)

    for j in range(P):
        pltpu.sync_copy(
            x_hbm.at[pl.ds(src[j], 1), pl.ds(104, 1), pl.ds(wb0, BW)],
            buf104.at[j])
    compute_channel(104,
                    lambda j, v: buf104[j, 0, 0, pl.ds(v * 16, 16)], 0)

    wait_out()
    wait_out()


@jax.jit
def _sc_call(x3, cwflat, cb):
    mesh = plsc.VectorSubcoreMesh(core_axis_name="c", subcore_axis_name="s")
    f = functools.partial(
        pl.kernel, mesh=mesh,
        compiler_params=pltpu.CompilerParams(needs_layout_passes=False),
        out_type=jax.ShapeDtypeStruct((7, 15, B), jnp.float32),
        scratch_types=[
            pltpu.VMEM((2, P, 1, 8, BW), jnp.float32),
            pltpu.VMEM((P, 1, 1, BW), jnp.float32),
            pltpu.VMEM((2, 1, BW), jnp.float32),
            pltpu.VMEM((C * 4,), jnp.float32),
            pltpu.VMEM((C,), jnp.float32),
            pltpu.VMEM((P, 1, 128), jnp.float32),
            pltpu.SemaphoreType.DMA,
            pltpu.SemaphoreType.DMA,
        ],
    )(_sc_body)
    return f(x3, cwflat, cb)


def kernel(augmented_tensor_temp, conv_weight, conv_bias):
    x3 = augmented_tensor_temp.transpose(2, 3, 1, 0).reshape(P, C, B)
    cwflat = conv_weight.reshape(-1)
    out3 = _sc_call(x3, cwflat, conv_bias)
    return out3.transpose(2, 1, 0)
